# baseline (speedup 1.0000x reference)
import functools

import jax
import jax.numpy as jnp
from jax import lax
from jax.experimental import pallas as pl
from jax.experimental.pallas import tpu as pltpu
from jax.experimental.pallas import tpu_sc as plsc

N = 1048576
NUM_CORES = 2
NUM_SUBCORES = 16
NW = NUM_CORES * NUM_SUBCORES
ROWS_PER_W = N // NW
CHUNK_ROWS = 4096
NUM_CHUNKS = ROWS_PER_W // CHUNK_ROWS
LANES = 16
G_CHUNK = 2 * CHUNK_ROWS // LANES


def _body(pos2_hbm, vel2_hbm, diam_hbm,
          dpos2_hbm, dvel2_hbm, ddiam_hbm,
          pos_v, vel_v, diam_v):
    wid = lax.axis_index("s") * NUM_CORES + lax.axis_index("c")

    lane = lax.iota(jnp.int32, LANES)
    ones_i = jnp.ones((LANES,), jnp.int32)
    half = lax.shift_right_logical(lane, ones_i)
    col = lane & ones_i
    pat = col.astype(jnp.float32) * jnp.full((LANES,), -20.0, jnp.float32)
    zeros = jnp.zeros((LANES,), jnp.float32)
    halves = jnp.full((LANES,), 0.5, jnp.float32)

    for c in range(NUM_CHUNKS):
        row0 = wid * ROWS_PER_W + c * CHUNK_ROWS
        pltpu.sync_copy(pos2_hbm.at[pl.ds(row0, CHUNK_ROWS)], pos_v)
        pltpu.sync_copy(vel2_hbm.at[pl.ds(row0, CHUNK_ROWS)], vel_v)
        pltpu.sync_copy(diam_hbm.at[pl.ds(row0, CHUNK_ROWS)], diam_v)

        def body(k, carry):
            rows = jnp.full((LANES,), k * (LANES // 2), jnp.int32) + half
            v = plsc.load_gather(vel_v, [rows, col])
            y = plsc.load_gather(pos_v, [rows, ones_i])
            d = plsc.load_gather(diam_v, [rows])
            m = y <= halves * d
            plsc.store_scatter(vel_v, [rows, col], jnp.where(m, zeros, v))
            plsc.store_scatter(pos_v, [rows, col], jnp.where(m, zeros, pat))
            plsc.store_scatter(diam_v, [rows], zeros)
            return carry
        lax.fori_loop(0, G_CHUNK, body, 0)

        pltpu.sync_copy(vel_v, dpos2_hbm.at[pl.ds(row0, CHUNK_ROWS)])
        pltpu.sync_copy(pos_v, dvel2_hbm.at[pl.ds(row0, CHUNK_ROWS)])
        pltpu.sync_copy(diam_v, ddiam_hbm.at[pl.ds(row0, CHUNK_ROWS)])


_sc_call = functools.partial(
    pl.kernel,
    out_type=(
        jax.ShapeDtypeStruct((N, 2), jnp.float32),
        jax.ShapeDtypeStruct((N, 2), jnp.float32),
        jax.ShapeDtypeStruct((N,), jnp.float32),
    ),
    mesh=plsc.VectorSubcoreMesh(core_axis_name="c", subcore_axis_name="s"),
    compiler_params=pltpu.CompilerParams(
        needs_layout_passes=False, use_tc_tiling_on_sc=False),
    scratch_types=[
        pltpu.VMEM((CHUNK_ROWS, 2), jnp.float32),
        pltpu.VMEM((CHUNK_ROWS, 2), jnp.float32),
        pltpu.VMEM((CHUNK_ROWS,), jnp.float32),
    ],
)(_body)


@jax.jit
def kernel(t, pos, vel, diameter):
    del t
    return _sc_call(pos, vel, diameter)


# speedup vs baseline: 59.8207x; 59.8207x over previous
import functools

import jax
import jax.numpy as jnp
from jax import lax
from jax.experimental import pallas as pl
from jax.experimental.pallas import tpu as pltpu
from jax.experimental.pallas import tpu_sc as plsc

N = 1048576
TILE = 128
NUM_CORES = 2
NUM_SUBCORES = 16
NW = NUM_CORES * NUM_SUBCORES
ROWS_PER_W = N // NW
CHUNK_ROWS = 4096
NUM_CHUNKS = ROWS_PER_W // CHUNK_ROWS
LANES = 16
G_CHUNK = CHUNK_ROWS // LANES


def _body(pos_hbm, vel_hbm, diam_hbm,
          dpos_hbm, dvel_hbm, ddiam_hbm,
          pos_v, vel_v, diam_v):
    wid = lax.axis_index("s") * NUM_CORES + lax.axis_index("c")

    zeros = jnp.zeros((LANES,), jnp.float32)
    neg20 = jnp.full((LANES,), -20.0, jnp.float32)
    halves = jnp.full((LANES,), 0.5, jnp.float32)

    for c in range(NUM_CHUNKS):
        row0 = wid * ROWS_PER_W + c * CHUNK_ROWS
        e0 = 2 * row0
        pltpu.sync_copy(pos_hbm.at[pl.ds(e0, 2 * CHUNK_ROWS)], pos_v)
        pltpu.sync_copy(vel_hbm.at[pl.ds(e0, 2 * CHUNK_ROWS)], vel_v)
        pltpu.sync_copy(diam_hbm.at[pl.ds(row0, CHUNK_ROWS)], diam_v)

        def body(g, carry):
            xb = (g // 8) * (2 * TILE) + (g % 8) * LANES
            yb = xb + TILE
            rb = g * LANES
            y = pos_v[pl.ds(yb, LANES)]
            d = diam_v[pl.ds(rb, LANES)]
            m = y <= halves * d
            vx = vel_v[pl.ds(xb, LANES)]
            vy = vel_v[pl.ds(yb, LANES)]
            vel_v[pl.ds(xb, LANES)] = jnp.where(m, zeros, vx)
            vel_v[pl.ds(yb, LANES)] = jnp.where(m, zeros, vy)
            pos_v[pl.ds(xb, LANES)] = zeros
            pos_v[pl.ds(yb, LANES)] = jnp.where(m, zeros, neg20)
            diam_v[pl.ds(rb, LANES)] = zeros
            return carry
        lax.fori_loop(0, G_CHUNK, body, 0)

        pltpu.sync_copy(vel_v, dpos_hbm.at[pl.ds(e0, 2 * CHUNK_ROWS)])
        pltpu.sync_copy(pos_v, dvel_hbm.at[pl.ds(e0, 2 * CHUNK_ROWS)])
        pltpu.sync_copy(diam_v, ddiam_hbm.at[pl.ds(row0, CHUNK_ROWS)])


_sc_call = functools.partial(
    pl.kernel,
    out_type=(
        jax.ShapeDtypeStruct((2 * N,), jnp.float32),
        jax.ShapeDtypeStruct((2 * N,), jnp.float32),
        jax.ShapeDtypeStruct((N,), jnp.float32),
    ),
    mesh=plsc.VectorSubcoreMesh(core_axis_name="c", subcore_axis_name="s"),
    compiler_params=pltpu.CompilerParams(
        needs_layout_passes=False, use_tc_tiling_on_sc=False),
    scratch_types=[
        pltpu.VMEM((2 * CHUNK_ROWS,), jnp.float32),
        pltpu.VMEM((2 * CHUNK_ROWS,), jnp.float32),
        pltpu.VMEM((CHUNK_ROWS,), jnp.float32),
    ],
)(_body)


def _to_flat(a):
    return jnp.transpose(a.reshape(N // TILE, TILE, 2), (0, 2, 1)).reshape(2 * N)


def _from_flat(f):
    return jnp.transpose(f.reshape(N // TILE, 2, TILE), (0, 2, 1)).reshape(N, 2)


@jax.jit
def kernel(t, pos, vel, diameter):
    del t
    dpos_f, dvel_f, ddiam = _sc_call(_to_flat(pos), _to_flat(vel), diameter)
    return (_from_flat(dpos_f), _from_flat(dvel_f), ddiam)


# speedup vs baseline: 69.2354x; 1.1574x over previous
import functools

import jax
import jax.numpy as jnp
from jax import lax
from jax.experimental import pallas as pl
from jax.experimental.pallas import tpu as pltpu
from jax.experimental.pallas import tpu_sc as plsc

N = 1048576
TILE = 128
NUM_CORES = 2
NUM_SUBCORES = 16
NW = NUM_CORES * NUM_SUBCORES
ROWS_PER_W = N // NW
CHUNK_ROWS = 4096
NUM_CHUNKS = ROWS_PER_W // CHUNK_ROWS
LANES = 16
G_CHUNK = CHUNK_ROWS // LANES


def _body(pos_hbm, vel_hbm, diam_hbm,
          dpos_hbm, dvel_hbm, ddiam_hbm,
          pos_v, vel_v, diam_v):
    wid = lax.axis_index("s") * NUM_CORES + lax.axis_index("c")

    zeros = jnp.zeros((LANES,), jnp.float32)
    neg20 = jnp.full((LANES,), -20.0, jnp.float32)
    halves = jnp.full((LANES,), 0.5, jnp.float32)

    for c in range(NUM_CHUNKS):
        row0 = wid * ROWS_PER_W + c * CHUNK_ROWS
        e0 = 2 * row0
        pltpu.sync_copy(pos_hbm.at[pl.ds(e0, 2 * CHUNK_ROWS)], pos_v)
        pltpu.sync_copy(vel_hbm.at[pl.ds(e0, 2 * CHUNK_ROWS)], vel_v)
        pltpu.sync_copy(diam_hbm.at[pl.ds(row0, CHUNK_ROWS)], diam_v)

        @plsc.parallel_loop(0, G_CHUNK, unroll=8)
        def body(g):
            xb = (g // 8) * (2 * TILE) + (g % 8) * LANES
            yb = xb + TILE
            rb = g * LANES
            y = pos_v[pl.ds(yb, LANES)]
            d = diam_v[pl.ds(rb, LANES)]
            m = y <= halves * d
            vx = vel_v[pl.ds(xb, LANES)]
            vy = vel_v[pl.ds(yb, LANES)]
            vel_v[pl.ds(xb, LANES)] = jnp.where(m, zeros, vx)
            vel_v[pl.ds(yb, LANES)] = jnp.where(m, zeros, vy)
            pos_v[pl.ds(xb, LANES)] = zeros
            pos_v[pl.ds(yb, LANES)] = jnp.where(m, zeros, neg20)
            diam_v[pl.ds(rb, LANES)] = zeros

        pltpu.sync_copy(vel_v, dpos_hbm.at[pl.ds(e0, 2 * CHUNK_ROWS)])
        pltpu.sync_copy(pos_v, dvel_hbm.at[pl.ds(e0, 2 * CHUNK_ROWS)])
        pltpu.sync_copy(diam_v, ddiam_hbm.at[pl.ds(row0, CHUNK_ROWS)])


_sc_call = functools.partial(
    pl.kernel,
    out_type=(
        jax.ShapeDtypeStruct((2 * N,), jnp.float32),
        jax.ShapeDtypeStruct((2 * N,), jnp.float32),
        jax.ShapeDtypeStruct((N,), jnp.float32),
    ),
    mesh=plsc.VectorSubcoreMesh(core_axis_name="c", subcore_axis_name="s"),
    compiler_params=pltpu.CompilerParams(
        needs_layout_passes=False, use_tc_tiling_on_sc=False),
    scratch_types=[
        pltpu.VMEM((2 * CHUNK_ROWS,), jnp.float32),
        pltpu.VMEM((2 * CHUNK_ROWS,), jnp.float32),
        pltpu.VMEM((CHUNK_ROWS,), jnp.float32),
    ],
)(_body)


def _to_flat(a):
    return jnp.transpose(a.reshape(N // TILE, TILE, 2), (0, 2, 1)).reshape(2 * N)


def _from_flat(f):
    return jnp.transpose(f.reshape(N // TILE, 2, TILE), (0, 2, 1)).reshape(N, 2)


@jax.jit
def kernel(t, pos, vel, diameter):
    del t
    dpos_f, dvel_f, ddiam = _sc_call(_to_flat(pos), _to_flat(vel), diameter)
    return (_from_flat(dpos_f), _from_flat(dvel_f), ddiam)


# speedup vs baseline: 87.5981x; 1.2652x over previous
import functools

import jax
import jax.numpy as jnp
from jax import lax
from jax.experimental import pallas as pl
from jax.experimental.pallas import tpu as pltpu
from jax.experimental.pallas import tpu_sc as plsc

N = 1048576
TILE = 128
NUM_CORES = 2
NUM_SUBCORES = 16
NW = NUM_CORES * NUM_SUBCORES
ROWS_PER_W = N // NW
CHUNK_ROWS = 4096
NUM_CHUNKS = ROWS_PER_W // CHUNK_ROWS
LANES = 16
G_CHUNK = CHUNK_ROWS // LANES


def _body(pos_hbm, vel_hbm, diam_hbm,
          dpos_hbm, dvel_hbm, ddiam_hbm,
          pos_v, vel_v, diam_v, sem_in, sem_out):
    wid = lax.axis_index("s") * NUM_CORES + lax.axis_index("c")

    zeros = jnp.zeros((LANES,), jnp.float32)
    neg20 = jnp.full((LANES,), -20.0, jnp.float32)
    halves = jnp.full((LANES,), 0.5, jnp.float32)

    def start_in(c, b):
        row0 = wid * ROWS_PER_W + c * CHUNK_ROWS
        e0 = 2 * row0
        return (
            pltpu.async_copy(
                pos_hbm.at[pl.ds(e0, 2 * CHUNK_ROWS)], pos_v.at[b], sem_in.at[b]),
            pltpu.async_copy(
                vel_hbm.at[pl.ds(e0, 2 * CHUNK_ROWS)], vel_v.at[b], sem_in.at[b]),
            pltpu.async_copy(
                diam_hbm.at[pl.ds(row0, CHUNK_ROWS)], diam_v.at[b], sem_in.at[b]),
        )

    def start_out(c, b):
        row0 = wid * ROWS_PER_W + c * CHUNK_ROWS
        e0 = 2 * row0
        return (
            pltpu.async_copy(
                vel_v.at[b], dpos_hbm.at[pl.ds(e0, 2 * CHUNK_ROWS)], sem_out.at[b]),
            pltpu.async_copy(
                pos_v.at[b], dvel_hbm.at[pl.ds(e0, 2 * CHUNK_ROWS)], sem_out.at[b]),
            pltpu.async_copy(
                diam_v.at[b], ddiam_hbm.at[pl.ds(row0, CHUNK_ROWS)], sem_out.at[b]),
        )

    in_flight = {0: start_in(0, 0)}
    out_flight = {}
    for c in range(NUM_CHUNKS):
        b = c % 2
        for h in in_flight.pop(c):
            h.wait()

        pos_b, vel_b, diam_b = pos_v.at[b], vel_v.at[b], diam_v.at[b]

        @plsc.parallel_loop(0, G_CHUNK, unroll=8)
        def body(g):
            xb = (g // 8) * (2 * TILE) + (g % 8) * LANES
            yb = xb + TILE
            rb = g * LANES
            y = pos_b[pl.ds(yb, LANES)]
            d = diam_b[pl.ds(rb, LANES)]
            m = y <= halves * d
            vx = vel_b[pl.ds(xb, LANES)]
            vy = vel_b[pl.ds(yb, LANES)]
            vel_b[pl.ds(xb, LANES)] = jnp.where(m, zeros, vx)
            vel_b[pl.ds(yb, LANES)] = jnp.where(m, zeros, vy)
            pos_b[pl.ds(xb, LANES)] = zeros
            pos_b[pl.ds(yb, LANES)] = jnp.where(m, zeros, neg20)
            diam_b[pl.ds(rb, LANES)] = zeros

        out_flight[c] = start_out(c, b)
        if c + 1 < NUM_CHUNKS:
            nb = (c + 1) % 2
            if c - 1 >= 0:
                for h in out_flight.pop(c - 1):
                    h.wait()
            in_flight[c + 1] = start_in(c + 1, nb)
    for c in list(out_flight):
        for h in out_flight.pop(c):
            h.wait()


_sc_call = functools.partial(
    pl.kernel,
    out_type=(
        jax.ShapeDtypeStruct((2 * N,), jnp.float32),
        jax.ShapeDtypeStruct((2 * N,), jnp.float32),
        jax.ShapeDtypeStruct((N,), jnp.float32),
    ),
    mesh=plsc.VectorSubcoreMesh(core_axis_name="c", subcore_axis_name="s"),
    compiler_params=pltpu.CompilerParams(
        needs_layout_passes=False, use_tc_tiling_on_sc=False),
    scratch_types=[
        pltpu.VMEM((2, 2 * CHUNK_ROWS), jnp.float32),
        pltpu.VMEM((2, 2 * CHUNK_ROWS), jnp.float32),
        pltpu.VMEM((2, CHUNK_ROWS), jnp.float32),
        pltpu.SemaphoreType.DMA((2,)),
        pltpu.SemaphoreType.DMA((2,)),
    ],
)(_body)


def _to_flat(a):
    return jnp.transpose(a.reshape(N // TILE, TILE, 2), (0, 2, 1)).reshape(2 * N)


def _from_flat(f):
    return jnp.transpose(f.reshape(N // TILE, 2, TILE), (0, 2, 1)).reshape(N, 2)


@jax.jit
def kernel(t, pos, vel, diameter):
    del t
    dpos_f, dvel_f, ddiam = _sc_call(_to_flat(pos), _to_flat(vel), diameter)
    return (_from_flat(dpos_f), _from_flat(dvel_f), ddiam)


# speedup vs baseline: 93.7684x; 1.0704x over previous
import functools

import jax
import jax.numpy as jnp
from jax import lax
from jax.experimental import pallas as pl
from jax.experimental.pallas import tpu as pltpu
from jax.experimental.pallas import tpu_sc as plsc

N = 1048576
TILE = 128
NUM_CORES = 2
NUM_SUBCORES = 16
NW = NUM_CORES * NUM_SUBCORES
ROWS_PER_W = N // NW
CHUNK_ROWS = 8192
NUM_CHUNKS = ROWS_PER_W // CHUNK_ROWS
LANES = 16
G_CHUNK = CHUNK_ROWS // LANES


def _body(pos_hbm, vel_hbm, diam_hbm,
          dpos_hbm, dvel_hbm, ddiam_hbm,
          pos_v, vel_v, diam_v, sem_in, sem_out):
    wid = lax.axis_index("s") * NUM_CORES + lax.axis_index("c")

    zeros = jnp.zeros((LANES,), jnp.float32)
    neg20 = jnp.full((LANES,), -20.0, jnp.float32)
    halves = jnp.full((LANES,), 0.5, jnp.float32)

    def start_in(c, b):
        row0 = wid * ROWS_PER_W + c * CHUNK_ROWS
        e0 = 2 * row0
        return (
            pltpu.async_copy(
                pos_hbm.at[pl.ds(e0, 2 * CHUNK_ROWS)], pos_v.at[b], sem_in.at[b]),
            pltpu.async_copy(
                vel_hbm.at[pl.ds(e0, 2 * CHUNK_ROWS)], vel_v.at[b], sem_in.at[b]),
            pltpu.async_copy(
                diam_hbm.at[pl.ds(row0, CHUNK_ROWS)], diam_v.at[b], sem_in.at[b]),
        )

    def start_out(c, b):
        row0 = wid * ROWS_PER_W + c * CHUNK_ROWS
        e0 = 2 * row0
        return (
            pltpu.async_copy(
                vel_v.at[b], dpos_hbm.at[pl.ds(e0, 2 * CHUNK_ROWS)], sem_out.at[b]),
            pltpu.async_copy(
                pos_v.at[b], dvel_hbm.at[pl.ds(e0, 2 * CHUNK_ROWS)], sem_out.at[b]),
            pltpu.async_copy(
                diam_v.at[b], ddiam_hbm.at[pl.ds(row0, CHUNK_ROWS)], sem_out.at[b]),
        )

    in_flight = {0: start_in(0, 0)}
    out_flight = {}
    for c in range(NUM_CHUNKS):
        b = c % 2
        for h in in_flight.pop(c):
            h.wait()

        pos_b, vel_b, diam_b = pos_v.at[b], vel_v.at[b], diam_v.at[b]

        @plsc.parallel_loop(0, G_CHUNK, unroll=8)
        def body(g):
            xb = (g // 8) * (2 * TILE) + (g % 8) * LANES
            yb = xb + TILE
            rb = g * LANES
            y = pos_b[pl.ds(yb, LANES)]
            d = diam_b[pl.ds(rb, LANES)]
            m = y <= halves * d
            vx = vel_b[pl.ds(xb, LANES)]
            vy = vel_b[pl.ds(yb, LANES)]
            vel_b[pl.ds(xb, LANES)] = jnp.where(m, zeros, vx)
            vel_b[pl.ds(yb, LANES)] = jnp.where(m, zeros, vy)
            pos_b[pl.ds(xb, LANES)] = zeros
            pos_b[pl.ds(yb, LANES)] = jnp.where(m, zeros, neg20)
            diam_b[pl.ds(rb, LANES)] = zeros

        out_flight[c] = start_out(c, b)
        if c + 1 < NUM_CHUNKS:
            nb = (c + 1) % 2
            if c - 1 >= 0:
                for h in out_flight.pop(c - 1):
                    h.wait()
            in_flight[c + 1] = start_in(c + 1, nb)
    for c in list(out_flight):
        for h in out_flight.pop(c):
            h.wait()


_sc_call = functools.partial(
    pl.kernel,
    out_type=(
        jax.ShapeDtypeStruct((2 * N,), jnp.float32),
        jax.ShapeDtypeStruct((2 * N,), jnp.float32),
        jax.ShapeDtypeStruct((N,), jnp.float32),
    ),
    mesh=plsc.VectorSubcoreMesh(core_axis_name="c", subcore_axis_name="s"),
    compiler_params=pltpu.CompilerParams(
        needs_layout_passes=False, use_tc_tiling_on_sc=False),
    scratch_types=[
        pltpu.VMEM((2, 2 * CHUNK_ROWS), jnp.float32),
        pltpu.VMEM((2, 2 * CHUNK_ROWS), jnp.float32),
        pltpu.VMEM((2, CHUNK_ROWS), jnp.float32),
        pltpu.SemaphoreType.DMA((2,)),
        pltpu.SemaphoreType.DMA((2,)),
    ],
)(_body)


def _to_flat(a):
    return jnp.transpose(a.reshape(N // TILE, TILE, 2), (0, 2, 1)).reshape(2 * N)


def _from_flat(f):
    return jnp.transpose(f.reshape(N // TILE, 2, TILE), (0, 2, 1)).reshape(N, 2)


@jax.jit
def kernel(t, pos, vel, diameter):
    del t
    dpos_f, dvel_f, ddiam = _sc_call(_to_flat(pos), _to_flat(vel), diameter)
    return (_from_flat(dpos_f), _from_flat(dvel_f), ddiam)


# speedup vs baseline: 101.3678x; 1.0810x over previous
import functools

import jax
import jax.numpy as jnp
from jax import lax
from jax.experimental import pallas as pl
from jax.experimental.pallas import tpu as pltpu
from jax.experimental.pallas import tpu_sc as plsc

N = 1048576
TILE = 128
NT = N // TILE
NUM_CORES = 2
NUM_SUBCORES = 16
NW = NUM_CORES * NUM_SUBCORES
ROWS_PER_W = N // NW
CHUNK_ROWS = 8192
NUM_CHUNKS = ROWS_PER_W // CHUNK_ROWS
T_CHUNK = CHUNK_ROWS // TILE
LANES = 16
G_CHUNK = CHUNK_ROWS // LANES


def _body(pos_hbm, vel_hbm, diam_hbm,
          dpos_hbm, dvel_hbm, ddiam_hbm,
          y_v, vel_v, diam_v, zero_v, sem_in, sem_out):
    wid = lax.axis_index("s") * NUM_CORES + lax.axis_index("c")
    pos3 = pos_hbm
    dvel3 = dvel_hbm

    zeros = jnp.zeros((LANES,), jnp.float32)
    neg20 = jnp.full((LANES,), -20.0, jnp.float32)
    halves = jnp.full((LANES,), 0.5, jnp.float32)

    @plsc.parallel_loop(0, T_CHUNK * (TILE // LANES), unroll=8)
    def zbody(g):
        zero_v[g // 8, pl.ds((g % 8) * LANES, LANES)] = zeros

    def start_in(c, b):
        t0 = wid * (ROWS_PER_W // TILE) + c * T_CHUNK
        r0 = t0 * TILE
        return (
            pltpu.async_copy(
                pos3.at[pl.ds(t0, T_CHUNK), 1], y_v.at[b], sem_in.at[b]),
            pltpu.async_copy(
                vel_hbm.at[pl.ds(2 * t0, 2 * T_CHUNK)], vel_v.at[b],
                sem_in.at[b]),
            pltpu.async_copy(
                diam_hbm.at[pl.ds(t0, T_CHUNK)], diam_v.at[b], sem_in.at[b]),
        )

    def start_out(c, b):
        t0 = wid * (ROWS_PER_W // TILE) + c * T_CHUNK
        return (
            pltpu.async_copy(
                vel_v.at[b], dpos_hbm.at[pl.ds(2 * t0, 2 * T_CHUNK)],
                sem_out.at[b]),
            pltpu.async_copy(
                y_v.at[b], dvel3.at[pl.ds(t0, T_CHUNK), 1], sem_out.at[b]),
            pltpu.async_copy(
                zero_v, dvel3.at[pl.ds(t0, T_CHUNK), 0], sem_out.at[b]),
            pltpu.async_copy(
                zero_v, ddiam_hbm.at[pl.ds(t0, T_CHUNK)], sem_out.at[b]),
        )

    in_flight = {0: start_in(0, 0)}
    out_flight = {}
    for c in range(NUM_CHUNKS):
        b = c % 2
        for h in in_flight.pop(c):
            h.wait()

        y_b, vel_b, diam_b = y_v.at[b], vel_v.at[b], diam_v.at[b]

        @plsc.parallel_loop(0, G_CHUNK, unroll=8)
        def body(g):
            t = g // 8
            s = (g % 8) * LANES
            y = y_b[t, pl.ds(s, LANES)]
            d = diam_b[t, pl.ds(s, LANES)]
            m = y <= halves * d
            vx = vel_b[2 * t, pl.ds(s, LANES)]
            vy = vel_b[2 * t + 1, pl.ds(s, LANES)]
            vel_b[2 * t, pl.ds(s, LANES)] = jnp.where(m, zeros, vx)
            vel_b[2 * t + 1, pl.ds(s, LANES)] = jnp.where(m, zeros, vy)
            y_b[t, pl.ds(s, LANES)] = jnp.where(m, zeros, neg20)

        out_flight[c] = start_out(c, b)
        if c + 1 < NUM_CHUNKS:
            nb = (c + 1) % 2
            if c - 1 >= 0:
                for h in out_flight.pop(c - 1):
                    h.wait()
            in_flight[c + 1] = start_in(c + 1, nb)
    for c in list(out_flight):
        for h in out_flight.pop(c):
            h.wait()


_sc_call = functools.partial(
    pl.kernel,
    out_type=(
        jax.ShapeDtypeStruct((2 * NT, TILE), jnp.float32),
        jax.ShapeDtypeStruct((NT, 2, TILE), jnp.float32),
        jax.ShapeDtypeStruct((NT, TILE), jnp.float32),
    ),
    mesh=plsc.VectorSubcoreMesh(core_axis_name="c", subcore_axis_name="s"),
    compiler_params=pltpu.CompilerParams(
        needs_layout_passes=False, use_tc_tiling_on_sc=False),
    scratch_types=[
        pltpu.VMEM((2, T_CHUNK, TILE), jnp.float32),
        pltpu.VMEM((2, 2 * T_CHUNK, TILE), jnp.float32),
        pltpu.VMEM((2, T_CHUNK, TILE), jnp.float32),
        pltpu.VMEM((T_CHUNK, TILE), jnp.float32),
        pltpu.SemaphoreType.DMA((2,)),
        pltpu.SemaphoreType.DMA((2,)),
    ],
)(_body)


def _to_view3(a):
    return jnp.transpose(a.reshape(N // TILE, TILE, 2), (0, 2, 1))


def _to_view2(a):
    return _to_view3(a).reshape(2 * NT, TILE)


def _from_view3(f):
    return jnp.transpose(f, (0, 2, 1)).reshape(N, 2)


@jax.jit
def kernel(t, pos, vel, diameter):
    del t
    dpos_f, dvel3, ddiam = _sc_call(
        _to_view3(pos), _to_view2(vel), diameter.reshape(NT, TILE))
    return (_from_view3(dpos_f.reshape(NT, 2, TILE)), _from_view3(dvel3),
            ddiam.reshape(N))
